# Initial kernel scaffold; baseline (speedup 1.0000x reference)
import jax
import jax.numpy as jnp
from jax.experimental import pallas as pl


def kernel(x, edge_index, W1, a1_src, a1_dst, b1, W2, a2_src, a2_dst, b2):
    raise NotImplementedError("write your pallas kernel here")


# speedup vs baseline: 49.1679x; 49.1679x over previous
import functools

import jax
import jax.numpy as jnp
import numpy as np
from jax import lax
from jax.experimental import pallas as pl
from jax.experimental.pallas import tpu as pltpu
from jax.experimental.pallas import tpu_sc as plsc

N = 10000
D = 128
E = 320000
H1 = 2
C1 = 8
OUT = 128

NP = 10240
GARBAGE = N
NC = 2
NS = 16
NW = NC * NS
G = 128
K = 81
KG = K * G
EP = NW * KG
ET = E + N
RPT = NP // NS

R1 = 32
R2 = 144
G2 = 96
K2 = KG // G2

BR = 512
NB = NP // BR

_f32 = jnp.float32
_i32 = jnp.int32


def _tc1_body(x_ref, w1_ref, a1_ref, s1_ref, t1_ref, alr_ref):
    xw = jnp.dot(x_ref[...], w1_ref[...], preferred_element_type=_f32)
    t1 = jnp.dot(xw, s1_ref[...], preferred_element_type=_f32)
    col = lax.broadcasted_iota(_i32, (BR, R1), 1)
    t1_ref[...] = t1 + jnp.where((col == C1) | (col == 2 * C1 + C1), 1.0, 0.0)
    alr_ref[...] = lax.dot_general(a1_ref[...], xw, (((1,), (1,)), ((), ())),
                                   preferred_element_type=_f32)


def _tc1(xp, W1, A1p, S1):
    return pl.pallas_call(
        _tc1_body,
        grid=(NB,),
        in_specs=[
            pl.BlockSpec((BR, D), lambda j: (j, 0)),
            pl.BlockSpec((D, H1 * C1), lambda j: (0, 0)),
            pl.BlockSpec((8, H1 * C1), lambda j: (0, 0)),
            pl.BlockSpec((H1 * C1, R1), lambda j: (0, 0)),
        ],
        out_specs=[
            pl.BlockSpec((BR, R1), lambda j: (j, 0)),
            pl.BlockSpec((8, BR), lambda j: (0, j)),
        ],
        out_shape=[
            jax.ShapeDtypeStruct((NP, R1), _f32),
            jax.ShapeDtypeStruct((8, NP), _f32),
        ],
    )(xp, W1, A1p, S1)


def _tc2_body(p_ref, b1_ref, w2_ref, a2_ref, s2_ref, t2_ref, alr_ref):
    s = p_ref[0] + p_ref[1]
    f0 = s[:, 0:C1] / (s[:, C1:C1 + 1] + 1e-16)
    f1 = s[:, 16:16 + C1] / (s[:, 16 + C1:16 + C1 + 1] + 1e-16)
    h = jnp.concatenate([f0, f1], axis=1) + b1_ref[0:1, :]
    h = jnp.maximum(h, 0.0)
    xw2 = jnp.dot(h, w2_ref[...], preferred_element_type=_f32)
    t2 = jnp.dot(xw2, s2_ref[...], preferred_element_type=_f32)
    col = lax.broadcasted_iota(_i32, (BR, R2), 1)
    t2_ref[...] = t2 + jnp.where(col == OUT, 1.0, 0.0)
    alr_ref[...] = lax.dot_general(a2_ref[...], xw2, (((1,), (1,)), ((), ())),
                                   preferred_element_type=_f32)


def _tc2(P1, b1t, W2, A2p, S2):
    return pl.pallas_call(
        _tc2_body,
        grid=(NB,),
        in_specs=[
            pl.BlockSpec((NC, BR, R1), lambda j: (0, j, 0)),
            pl.BlockSpec((8, H1 * C1), lambda j: (0, 0)),
            pl.BlockSpec((H1 * C1, OUT), lambda j: (0, 0)),
            pl.BlockSpec((8, OUT), lambda j: (0, 0)),
            pl.BlockSpec((OUT, R2), lambda j: (0, 0)),
        ],
        out_specs=[
            pl.BlockSpec((BR, R2), lambda j: (j, 0)),
            pl.BlockSpec((8, BR), lambda j: (0, j)),
        ],
        out_shape=[
            jax.ShapeDtypeStruct((NP, R2), _f32),
            jax.ShapeDtypeStruct((8, NP), _f32),
        ],
    )(P1, b1t, W2, A2p, S2)


BR3 = 400


def _tc3_body(p_ref, b2_ref, o_ref):
    s = p_ref[0] + p_ref[1]
    o_ref[...] = s[:, 0:OUT] / (s[:, OUT:OUT + 1] + 1e-16) + b2_ref[0:1, :]


def _tc3(P2, b2t):
    return pl.pallas_call(
        _tc3_body,
        grid=(N // BR3,),
        in_specs=[
            pl.BlockSpec((NC, BR3, R2), lambda j: (0, j, 0)),
            pl.BlockSpec((8, OUT), lambda j: (0, 0)),
        ],
        out_specs=pl.BlockSpec((BR3, OUT), lambda j: (j, 0)),
        out_shape=jax.ShapeDtypeStruct((N, OUT), _f32),
    )(P2, b2t)


def _leaky(v):
    return jnp.where(v >= 0, v, 0.2 * v)


def _make_sc1():
    mesh = plsc.VectorSubcoreMesh(core_axis_name="c", subcore_axis_name="s")

    @functools.partial(
        pl.kernel,
        mesh=mesh,
        out_type=jax.ShapeDtypeStruct((NC, NP, R1), _f32),
        scratch_types=[
            pltpu.VMEM((NP,), _f32),
            pltpu.VMEM((NP,), _f32),
            pltpu.VMEM((NP,), _f32),
            pltpu.VMEM((NP,), _f32),
            pltpu.VMEM((K, G), _i32),
            pltpu.VMEM((K, G), _i32),
            pltpu.VMEM((G, R1), _f32),
            pltpu.VMEM((G,), _f32),
            pltpu.VMEM((G,), _f32),
            pltpu.VMEM_SHARED((NP, R1), _f32),
            pltpu.SemaphoreType.DMA,
        ],
        compiler_params=pltpu.CompilerParams(needs_layout_passes=False, use_tc_tiling_on_sc=False),
    )
    def sc1(srcw, dstw, alr, t1, outp,
            al0, al1, ar0, ar1, srcv, dstv, gbuf, w0b, w1b, accs, sem):
        c = lax.axis_index("c")
        s = lax.axis_index("s")
        wid = s * NC + c
        pltpu.sync_copy(alr.at[0], al0)
        pltpu.sync_copy(alr.at[1], al1)
        pltpu.sync_copy(alr.at[2], ar0)
        pltpu.sync_copy(alr.at[3], ar1)
        pltpu.sync_copy(srcw.at[wid], srcv)
        pltpu.sync_copy(dstw.at[wid], dstv)
        for g in range(G):
            for k in range(R1 // 16):
                gbuf[g, pl.ds(k * 16, 16)] = jnp.zeros((16,), _f32)
        for t in range(RPT // G):
            pltpu.sync_copy(gbuf, accs.at[pl.ds(s * RPT + t * G, G)])
        plsc.subcore_barrier()

        def chunk(ci, carry):
            cp = pltpu.async_copy(t1.at[srcv.at[ci]], gbuf, sem)
            for j in range(G // 16):
                s16 = srcv[ci, pl.ds(j * 16, 16)]
                d16 = dstv[ci, pl.ds(j * 16, 16)]
                e0 = _leaky(plsc.load_gather(al0, [s16]) +
                            plsc.load_gather(ar0, [d16]))
                e1 = _leaky(plsc.load_gather(al1, [s16]) +
                            plsc.load_gather(ar1, [d16]))
                w0b[pl.ds(j * 16, 16)] = jnp.exp(e0)
                w1b[pl.ds(j * 16, 16)] = jnp.exp(e1)
            cp.wait()
            for g in range(G):
                gi = jnp.full((16,), g, _i32)
                ws0 = plsc.load_gather(w0b, [gi])
                ws1 = plsc.load_gather(w1b, [gi])
                gbuf[g, pl.ds(0, 16)] = gbuf[g, pl.ds(0, 16)] * ws0
                gbuf[g, pl.ds(16, 16)] = gbuf[g, pl.ds(16, 16)] * ws1
            pltpu.sync_copy(gbuf, accs.at[dstv.at[ci]], add=True)
            return carry

        lax.fori_loop(0, K, chunk, 0)
        plsc.subcore_barrier()
        for t in range(RPT // G):
            pltpu.sync_copy(accs.at[pl.ds(s * RPT + t * G, G)], gbuf)
            pltpu.sync_copy(gbuf, outp.at[c, pl.ds(s * RPT + t * G, G)])

    return sc1


def _make_sc2():
    mesh = plsc.VectorSubcoreMesh(core_axis_name="c", subcore_axis_name="s")

    @functools.partial(
        pl.kernel,
        mesh=mesh,
        out_type=jax.ShapeDtypeStruct((NC, NP, R2), _f32),
        scratch_types=[
            pltpu.VMEM((NP,), _f32),
            pltpu.VMEM((NP,), _f32),
            pltpu.VMEM((2, G2), _i32),
            pltpu.VMEM((G2, R2), _f32),
            pltpu.VMEM((G2,), _f32),
            pltpu.VMEM_SHARED((NP, R2), _f32),
            pltpu.SemaphoreType.DMA,
        ],
        compiler_params=pltpu.CompilerParams(needs_layout_passes=False, use_tc_tiling_on_sc=False),
    )
    def sc2(sidx, alr, t2, outp,
            alv, arv, sb, gbuf, wb, accs, sem):
        c = lax.axis_index("c")
        s = lax.axis_index("s")
        wid = s * NC + c
        pltpu.sync_copy(alr.at[0], alv)
        pltpu.sync_copy(alr.at[1], arv)
        for g in range(80):
            for k in range(R2 // 16):
                gbuf[g, pl.ds(k * 16, 16)] = jnp.zeros((16,), _f32)
        for t in range(RPT // 80):
            pltpu.sync_copy(gbuf.at[pl.ds(0, 80)],
                            accs.at[pl.ds(s * RPT + t * 80, 80)])
        plsc.subcore_barrier()

        def chunk(ci, carry):
            pltpu.sync_copy(sidx.at[wid, ci], sb)
            cp = pltpu.async_copy(t2.at[sb.at[0]], gbuf, sem)
            for j in range(G2 // 16):
                s16 = sb[0, pl.ds(j * 16, 16)]
                d16 = sb[1, pl.ds(j * 16, 16)]
                e = _leaky(plsc.load_gather(alv, [s16]) +
                           plsc.load_gather(arv, [d16]))
                wb[pl.ds(j * 16, 16)] = jnp.exp(e)
            cp.wait()
            for g in range(G2):
                ws = plsc.load_gather(wb, [jnp.full((16,), g, _i32)])
                for k in range(R2 // 16):
                    gbuf[g, pl.ds(k * 16, 16)] = gbuf[g, pl.ds(k * 16, 16)] * ws
            pltpu.sync_copy(gbuf, accs.at[sb.at[1]], add=True)
            return carry

        lax.fori_loop(0, K2, chunk, 0)
        plsc.subcore_barrier()
        for t in range(RPT // 80):
            pltpu.sync_copy(accs.at[pl.ds(s * RPT + t * 80, 80)],
                            gbuf.at[pl.ds(0, 80)])
            pltpu.sync_copy(gbuf.at[pl.ds(0, 80)],
                            outp.at[c, pl.ds(s * RPT + t * 80, 80)])

    return sc2


_sc1 = _make_sc1()
_sc2 = _make_sc2()


def kernel(x, edge_index, W1, a1_src, a1_dst, b1, W2, a2_src, a2_dst, b2):
    xp = jnp.pad(x, ((0, NP - N), (0, 0)))

    z8 = jnp.zeros((H1 * C1,), _f32)
    A1p = jnp.stack([
        jnp.concatenate([a1_src[0], jnp.zeros((C1,), _f32)]),
        jnp.concatenate([jnp.zeros((C1,), _f32), a1_src[1]]),
        jnp.concatenate([a1_dst[0], jnp.zeros((C1,), _f32)]),
        jnp.concatenate([jnp.zeros((C1,), _f32), a1_dst[1]]),
        z8, z8, z8, z8,
    ])
    S1np = np.zeros((H1 * C1, R1), np.float32)
    S1np[np.arange(C1), np.arange(C1)] = 1.0
    S1np[C1 + np.arange(C1), 16 + np.arange(C1)] = 1.0
    S1 = jnp.asarray(S1np)

    A2p = jnp.concatenate([a2_src, a2_dst, jnp.zeros((6, OUT), _f32)], axis=0)
    S2np = np.zeros((OUT, R2), np.float32)
    S2np[np.arange(OUT), np.arange(OUT)] = 1.0
    S2 = jnp.asarray(S2np)

    b1t = jnp.broadcast_to(b1, (8, H1 * C1))
    b2t = jnp.broadcast_to(b2, (8, OUT))

    loop = jnp.arange(N, dtype=_i32)
    pad = EP - ET
    esrc = jnp.concatenate([edge_index[0].astype(_i32), loop,
                            jnp.zeros((pad,), _i32)])
    edst = jnp.concatenate([edge_index[1].astype(_i32), loop,
                            jnp.full((pad,), GARBAGE, _i32)])
    srcw = esrc.reshape(NW, K, G)
    dstw = edst.reshape(NW, K, G)
    sidx2 = jnp.stack([esrc.reshape(NW, K2, G2),
                       edst.reshape(NW, K2, G2)], axis=2)

    T1, ALR1 = _tc1(xp, W1, A1p, S1)
    P1 = _sc1(srcw, dstw, ALR1, T1)
    T2, ALR2 = _tc2(P1, b1t, W2, A2p, S2)
    P2 = _sc2(sidx2, ALR2, T2)
    return _tc3(P2, b2t)
